# Initial kernel scaffold; baseline (speedup 1.0000x reference)
import jax
import jax.numpy as jnp
from jax.experimental import pallas as pl


def kernel(seq1, adj, W1_0, b1_0, g1_0, be1_0, W2_0, b2_0, g_0, be_0, W1_1, b1_1, g1_1, be1_1, W2_1, b2_1, g_1, be_1, Wp, bp):
    raise NotImplementedError("write your pallas kernel here")


# speedup vs baseline: 1.0059x; 1.0059x over previous
import jax
import jax.numpy as jnp
from jax.experimental import pallas as pl
from jax.experimental.pallas import tpu as pltpu

N = 10000
D = 128
H = 128
BM = 400


def _layer_body(adj_ref, h_ref, w1_ref, s1_ref, w2_ref, s2_ref, out_ref):
    pooled = jnp.dot(adj_ref[...].astype(jnp.bfloat16), h_ref[...],
                     preferred_element_type=jnp.float32)
    t = jnp.maximum(
        jnp.dot(pooled, w1_ref[...], preferred_element_type=jnp.float32)
        + s1_ref[...], 0.0)
    h2 = jnp.maximum(
        jnp.dot(t, w2_ref[...], preferred_element_type=jnp.float32)
        + s2_ref[...], 0.0)
    out_ref[...] = h2.astype(out_ref.dtype)


def _layer_proj_body(adj_ref, h_ref, w1_ref, s1_ref, w2_ref, s2_ref,
                     wp_ref, bp_ref, out_ref):
    pooled = jnp.dot(adj_ref[...].astype(jnp.bfloat16), h_ref[...],
                     preferred_element_type=jnp.float32)
    t = jnp.maximum(
        jnp.dot(pooled, w1_ref[...], preferred_element_type=jnp.float32)
        + s1_ref[...], 0.0)
    h2 = jnp.maximum(
        jnp.dot(t, w2_ref[...], preferred_element_type=jnp.float32)
        + s2_ref[...], 0.0)
    out_ref[...] = (jnp.dot(h2, wp_ref[...], preferred_element_type=jnp.float32)
                    + bp_ref[...])


def _const(shape):
    return pl.BlockSpec(shape, lambda i: (0,) * len(shape))


def _layer_call(adj, h_bf16, w1, s1, w2, s2):
    return pl.pallas_call(
        _layer_body,
        grid=(N // BM,),
        in_specs=[
            pl.BlockSpec((BM, N), lambda i: (i, 0)),
            _const((N, H)),
            _const((H, H)),
            _const((1, H)),
            _const((H, H)),
            _const((1, H)),
        ],
        out_specs=pl.BlockSpec((BM, H), lambda i: (i, 0)),
        out_shape=jax.ShapeDtypeStruct((N, H), jnp.bfloat16),
        compiler_params=pltpu.CompilerParams(
            dimension_semantics=("parallel",)),
    )(adj, h_bf16, w1, s1, w2, s2)


def _layer_proj_call(adj, h_bf16, w1, s1, w2, s2, wp, bp):
    return pl.pallas_call(
        _layer_proj_body,
        grid=(N // BM,),
        in_specs=[
            pl.BlockSpec((BM, N), lambda i: (i, 0)),
            _const((N, H)),
            _const((H, H)),
            _const((1, H)),
            _const((H, H)),
            _const((1, H)),
            _const((H, 1)),
            _const((1, 1)),
        ],
        out_specs=pl.BlockSpec((BM, 1), lambda i: (i, 0)),
        out_shape=jax.ShapeDtypeStruct((N, 1), jnp.float32),
        compiler_params=pltpu.CompilerParams(
            dimension_semantics=("parallel",)),
    )(adj, h_bf16, w1, s1, w2, s2, wp, bp)


def _fold_bn(W1, b1, g1, be1, W2, b2, g, be):
    inv = 1.0 / jnp.sqrt(1.0 + 1e-5)
    sc1 = g1 * inv
    sc2 = g * inv
    w1 = W1 * sc1[None, :]
    s1 = (b1 * sc1 + be1)[None, :]
    w2 = W2 * sc2[None, :]
    s2 = (b2 * sc2 + be)[None, :]
    return w1, s1, w2, s2


def kernel(seq1, adj, W1_0, b1_0, g1_0, be1_0, W2_0, b2_0, g_0, be_0,
           W1_1, b1_1, g1_1, be1_1, W2_1, b2_1, g_1, be_1, Wp, bp):
    w1a, s1a, w2a, s2a = _fold_bn(W1_0, b1_0, g1_0, be1_0, W2_0, b2_0, g_0, be_0)
    w1b, s1b, w2b, s2b = _fold_bn(W1_1, b1_1, g1_1, be1_1, W2_1, b2_1, g_1, be_1)
    h0 = seq1.astype(jnp.bfloat16)
    h1 = _layer_call(adj, h0, w1a, s1a, w2a, s2a)
    return _layer_proj_call(adj, h1, w1b, s1b, w2b, s2b,
                            Wp, bp.reshape(1, 1))
